# Initial kernel scaffold; baseline (speedup 1.0000x reference)
import jax
import jax.numpy as jnp
from jax.experimental import pallas as pl


def kernel(gt_boxes, pr_boxes):
    raise NotImplementedError("write your pallas kernel here")


# speedup vs baseline: 1234.1846x; 1234.1846x over previous
import functools

import jax
import jax.numpy as jnp
from jax.experimental import pallas as pl
from jax.experimental.pallas import tpu as pltpu

NC = 80


def _corners(cx, cy, w, h):
    x1 = cx - w / 2
    y1 = cy - h / 2
    x2 = cx + w / 2
    y2 = cy + h / 2
    return y1, x1, y2, x2


def _inter_union(g_cx, g_cy, g_w, g_h, p_cx, p_cy, p_w, p_h):
    gy1, gx1, gy2, gx2 = _corners(g_cx, g_cy, g_w, g_h)
    py1, px1, py2, px2 = _corners(p_cx, p_cy, p_w, p_h)
    in_ymin = jnp.maximum(gy1, py1)
    in_xmin = jnp.maximum(gx1, px1)
    in_ymax = jnp.minimum(gy2, py2)
    in_xmax = jnp.minimum(gx2, px2)
    in_w = jnp.maximum(0.0, in_xmax - in_xmin)
    in_h = jnp.maximum(0.0, in_ymax - in_ymin)
    inter = in_w * in_h
    areas = (g_w * g_h) + (p_w * p_h)
    union = areas - inter
    return inter, union


def _argmax_kernel(gt_ref, pr0_ref, best_ref, m_ref, a_ref, *, blk, num_pr, pb_steps):
    pb = pl.program_id(1)
    gt = gt_ref[0]
    ng = gt.shape[1]
    g_cx = gt[0:1, :]
    g_cy = gt[1:2, :]
    g_w = gt[2:3, :]
    g_h = gt[3:4, :]
    valid = g_cx != -1.0
    p = pr0_ref[...]
    p_cx = p[:, 0:1]
    p_cy = p[:, 1:2]
    p_w = p[:, 2:3]
    p_h = p[:, 3:4]
    inter, union = _inter_union(g_cx, g_cy, g_w, g_h, p_cx, p_cy, p_w, p_h)
    iou = inter / (union + 1e-5)
    iou = jnp.where(valid, iou, 0.0)

    @pl.when(pb == 0)
    def _():
        m_ref[...] = jnp.full((1, ng), -1.0, jnp.float32)
        a_ref[...] = jnp.zeros((1, ng), jnp.int32)

    bmax = jnp.max(iou, axis=0, keepdims=True)
    pidx = jax.lax.broadcasted_iota(jnp.int32, (blk, ng), 0) + pb * blk
    barg = jnp.min(jnp.where(iou == bmax, pidx, num_pr), axis=0, keepdims=True)
    better = bmax > m_ref[...]
    a_ref[...] = jnp.where(better, barg, a_ref[...])
    m_ref[...] = jnp.where(better, bmax, m_ref[...])

    @pl.when(pb == pb_steps - 1)
    def _():
        best_ref[0] = a_ref[...]


def _assign_kernel(gt_ref, pr0_ref, prb_ref, best_ref, cls_ref, loc_ref, msk_ref,
                   *, blk):
    pb = pl.program_id(1)
    gt = gt_ref[0]
    ng = gt.shape[1]
    g_cx = gt[0:1, :]
    g_cy = gt[1:2, :]
    g_w = gt[2:3, :]
    g_h = gt[3:4, :]
    g_cls = gt[4:5, :]
    g_conf = gt[5:6, :]
    valid = g_cx != -1.0

    p0 = pr0_ref[...]
    inter, union = _inter_union(
        g_cx, g_cy, g_w, g_h, p0[:, 0:1], p0[:, 1:2], p0[:, 2:3], p0[:, 3:4])
    ue = union + 1e-5
    thr = (inter >= 0.5 * ue) & valid
    ign = (inter >= 0.4 * ue) & (inter < 0.5 * ue) & valid

    pidx = jax.lax.broadcasted_iota(jnp.int32, (blk, ng), 0) + pb * blk
    best = best_ref[0]
    is_best = (pidx == best) & (g_conf > 0.0)

    g_iota = jax.lax.broadcasted_iota(jnp.int32, (blk, ng), 1)
    score = jnp.where(is_best, g_iota + ng, jnp.where(thr, g_iota, -1))
    smax = jnp.max(score, axis=1, keepdims=True)
    matched = smax >= 0
    sel = (score == smax) & matched
    cls_true = jnp.sum(jnp.where(sel, g_cls, 0.0), axis=1, keepdims=True)
    cls_true = jnp.where(matched, cls_true, float(NC))

    pb_attrs = prb_ref[0]
    b_cx = pb_attrs[:, 0:1]
    b_cy = pb_attrs[:, 1:2]
    b_w = pb_attrs[:, 2:3]
    b_h = pb_attrs[:, 3:4]
    cnt = thr.astype(jnp.float32) + is_best.astype(jnp.float32)
    lgw = jnp.log(jnp.where(valid, g_w, 1.0))
    lgh = jnp.log(jnp.where(valid, g_h, 1.0))
    s_cnt = jnp.sum(cnt, axis=1, keepdims=True)
    s_cx = jnp.sum(cnt * g_cx, axis=1, keepdims=True)
    s_cy = jnp.sum(cnt * g_cy, axis=1, keepdims=True)
    s_lw = jnp.sum(cnt * lgw, axis=1, keepdims=True)
    s_lh = jnp.sum(cnt * lgh, axis=1, keepdims=True)
    l0 = (s_cx - b_cx * s_cnt) / b_w
    l1 = (s_cy - b_cy * s_cnt) / b_h
    l2 = s_lw - s_cnt * jnp.log(b_w)
    l3 = s_lh - s_cnt * jnp.log(b_h)
    loc_ref[0] = jnp.concatenate([l0, l1, l2, l3], axis=1)

    bg = (cls_true == float(NC)).astype(jnp.float32)
    ignore_any = jnp.max(ign.astype(jnp.int32), axis=1, keepdims=True) > 0
    msk_ref[0] = jnp.where(ignore_any, -1.0, bg)

    c_iota = jax.lax.broadcasted_iota(jnp.int32, (blk, NC), 1)
    cls_ref[0] = (c_iota == cls_true.astype(jnp.int32)).astype(jnp.float32)


@jax.jit
def kernel(gt_boxes, pr_boxes):
    B, NG, _ = gt_boxes.shape
    _, NP, _ = pr_boxes.shape
    blk = 2000
    pb_steps = NP // blk

    gt_t = jnp.transpose(gt_boxes, (0, 2, 1))
    pr0 = pr_boxes[0]

    best = pl.pallas_call(
        functools.partial(_argmax_kernel, blk=blk, num_pr=NP, pb_steps=pb_steps),
        grid=(B, pb_steps),
        in_specs=[
            pl.BlockSpec((1, 6, NG), lambda b, p: (b, 0, 0)),
            pl.BlockSpec((blk, 4), lambda b, p: (p, 0)),
        ],
        out_specs=pl.BlockSpec((1, 1, NG), lambda b, p: (b, 0, 0)),
        out_shape=jax.ShapeDtypeStruct((B, 1, NG), jnp.int32),
        scratch_shapes=[
            pltpu.VMEM((1, NG), jnp.float32),
            pltpu.VMEM((1, NG), jnp.int32),
        ],
    )(gt_t, pr0)

    cls_out, loc_true, amask = pl.pallas_call(
        functools.partial(_assign_kernel, blk=blk),
        grid=(B, pb_steps),
        in_specs=[
            pl.BlockSpec((1, 6, NG), lambda b, p: (b, 0, 0)),
            pl.BlockSpec((blk, 4), lambda b, p: (p, 0)),
            pl.BlockSpec((1, blk, 4), lambda b, p: (b, p, 0)),
            pl.BlockSpec((1, 1, NG), lambda b, p: (b, 0, 0)),
        ],
        out_specs=[
            pl.BlockSpec((1, blk, NC), lambda b, p: (b, p, 0)),
            pl.BlockSpec((1, blk, 4), lambda b, p: (b, p, 0)),
            pl.BlockSpec((1, blk, 1), lambda b, p: (b, p, 0)),
        ],
        out_shape=[
            jax.ShapeDtypeStruct((B, NP, NC), jnp.float32),
            jax.ShapeDtypeStruct((B, NP, 4), jnp.float32),
            jax.ShapeDtypeStruct((B, NP, 1), jnp.float32),
        ],
    )(gt_t, pr0, pr_boxes, best)

    return (cls_out, loc_true, amask)


# speedup vs baseline: 2368.2393x; 1.9189x over previous
import functools

import jax
import jax.numpy as jnp
from jax.experimental import pallas as pl
from jax.experimental.pallas import tpu as pltpu

NC = 80


def _corners(cx, cy, w, h):
    x1 = cx - w / 2
    y1 = cy - h / 2
    x2 = cx + w / 2
    y2 = cy + h / 2
    return y1, x1, y2, x2


def _inter_union(g_cx, g_cy, g_w, g_h, p_cx, p_cy, p_w, p_h):
    gy1, gx1, gy2, gx2 = _corners(g_cx, g_cy, g_w, g_h)
    py1, px1, py2, px2 = _corners(p_cx, p_cy, p_w, p_h)
    in_ymin = jnp.maximum(gy1, py1)
    in_xmin = jnp.maximum(gx1, px1)
    in_ymax = jnp.minimum(gy2, py2)
    in_xmax = jnp.minimum(gx2, px2)
    in_w = jnp.maximum(0.0, in_xmax - in_xmin)
    in_h = jnp.maximum(0.0, in_ymax - in_ymin)
    inter = in_w * in_h
    areas = (g_w * g_h) + (p_w * p_h)
    union = areas - inter
    return inter, union


def _split_gt(gt):
    g_cx = gt[:, 0:1]
    g_cy = gt[:, 1:2]
    g_w = gt[:, 2:3]
    g_h = gt[:, 3:4]
    return g_cx, g_cy, g_w, g_h


def _argmax_kernel(gt_ref, pr0_ref, best_ref, m_ref, a_ref, *, blk, num_pr,
                   pb_steps):
    pb = pl.program_id(1)
    gt = gt_ref[0]
    ng = gt.shape[0]
    g_cx, g_cy, g_w, g_h = _split_gt(gt)
    valid = g_cx != -1.0
    p0 = pr0_ref[...]
    inter, union = _inter_union(g_cx, g_cy, g_w, g_h,
                                p0[0:1, :], p0[1:2, :], p0[2:3, :], p0[3:4, :])
    iou = inter / (union + 1e-5)
    iou = jnp.where(valid, iou, 0.0)

    @pl.when(pb == 0)
    def _():
        m_ref[...] = jnp.full((ng, 1), -1.0, jnp.float32)
        a_ref[...] = jnp.zeros((ng, 1), jnp.int32)

    bmax = jnp.max(iou, axis=1, keepdims=True)
    pidx = jax.lax.broadcasted_iota(jnp.int32, iou.shape, 1) + pb * blk
    barg = jnp.min(jnp.where(iou == bmax, pidx, num_pr), axis=1, keepdims=True)
    better = bmax > m_ref[...]
    a_ref[...] = jnp.where(better, barg, a_ref[...])
    m_ref[...] = jnp.where(better, bmax, m_ref[...])

    @pl.when(pb == pb_steps - 1)
    def _():
        best_ref[0] = a_ref[...]


def _assign_kernel(gt_ref, pr0_ref, prb_ref, best_ref, packed_ref, *, blk):
    pb = pl.program_id(1)
    gt = gt_ref[0]
    ng = gt.shape[0]
    g_cx, g_cy, g_w, g_h = _split_gt(gt)
    g_cls = gt[:, 4:5]
    g_conf = gt[:, 5:6]
    valid = g_cx != -1.0

    p0 = pr0_ref[...]
    inter, union = _inter_union(g_cx, g_cy, g_w, g_h,
                                p0[0:1, :], p0[1:2, :], p0[2:3, :], p0[3:4, :])
    ue = union + 1e-5
    thr = (inter >= 0.5 * ue) & valid
    ign = (inter >= 0.4 * ue) & (inter < 0.5 * ue) & valid

    pidx = jax.lax.broadcasted_iota(jnp.int32, inter.shape, 1) + pb * blk
    best = best_ref[0]
    is_best = (pidx == best) & (g_conf > 0.0)

    g_iota = jax.lax.broadcasted_iota(jnp.int32, inter.shape, 0)
    score = jnp.where(is_best, g_iota + ng, jnp.where(thr, g_iota, -1))
    smax = jnp.max(score, axis=0, keepdims=True)
    matched = smax >= 0
    sel = (score == smax) & matched
    cls_true = jnp.sum(jnp.where(sel, g_cls, 0.0), axis=0, keepdims=True)
    cls_true = jnp.where(matched, cls_true, float(NC))

    prb = prb_ref[0]
    b_cx = prb[0:1, :]
    b_cy = prb[1:2, :]
    b_w = prb[2:3, :]
    b_h = prb[3:4, :]
    cnt = thr.astype(jnp.float32) + is_best.astype(jnp.float32)
    lgw = jnp.log(jnp.where(valid, g_w, 1.0))
    lgh = jnp.log(jnp.where(valid, g_h, 1.0))
    s_cnt = jnp.sum(cnt, axis=0, keepdims=True)
    s_cx = jnp.sum(cnt * g_cx, axis=0, keepdims=True)
    s_cy = jnp.sum(cnt * g_cy, axis=0, keepdims=True)
    s_lw = jnp.sum(cnt * lgw, axis=0, keepdims=True)
    s_lh = jnp.sum(cnt * lgh, axis=0, keepdims=True)
    bw_safe = jnp.maximum(b_w, 1e-20)
    bh_safe = jnp.maximum(b_h, 1e-20)
    l0 = (s_cx - b_cx * s_cnt) / bw_safe
    l1 = (s_cy - b_cy * s_cnt) / bh_safe
    l2 = s_lw - s_cnt * jnp.log(bw_safe)
    l3 = s_lh - s_cnt * jnp.log(bh_safe)

    bg = (cls_true == float(NC)).astype(jnp.float32)
    ignore_any = jnp.max(ign.astype(jnp.int32), axis=0, keepdims=True) > 0
    amask = jnp.where(ignore_any, -1.0, bg)

    zeros2 = jnp.zeros((2,) + cls_true.shape[1:], jnp.float32)
    packed_ref[0] = jnp.concatenate(
        [cls_true, l0, l1, l2, l3, amask, zeros2], axis=0)


def _emit_kernel(packed_ref, cls_ref, loc_ref, msk_ref):
    d = packed_ref[...]
    cls_true = d[:, 0:1].astype(jnp.int32)
    c_iota = jax.lax.broadcasted_iota(jnp.int32, (d.shape[0], NC), 1)
    cls_ref[...] = (c_iota == cls_true).astype(jnp.float32)
    loc_ref[...] = d[:, 1:5]
    msk_ref[...] = d[:, 5:6]


@jax.jit
def kernel(gt_boxes, pr_boxes):
    B, NG, _ = gt_boxes.shape
    _, NP, _ = pr_boxes.shape
    blk = 2048
    npad = -NP % blk
    NPP = NP + npad
    pb_steps = NPP // blk

    pr_t = jnp.transpose(pr_boxes, (0, 2, 1))
    pr_t = jnp.pad(pr_t, ((0, 0), (0, 0), (0, npad)))
    pr0_t = pr_t[0]

    best = pl.pallas_call(
        functools.partial(_argmax_kernel, blk=blk, num_pr=NPP,
                          pb_steps=pb_steps),
        grid=(B, pb_steps),
        in_specs=[
            pl.BlockSpec((1, NG, 6), lambda b, p: (b, 0, 0)),
            pl.BlockSpec((4, blk), lambda b, p: (0, p)),
        ],
        out_specs=pl.BlockSpec((1, NG, 1), lambda b, p: (b, 0, 0)),
        out_shape=jax.ShapeDtypeStruct((B, NG, 1), jnp.int32),
        scratch_shapes=[
            pltpu.VMEM((NG, 1), jnp.float32),
            pltpu.VMEM((NG, 1), jnp.int32),
        ],
    )(gt_boxes, pr0_t)

    packed = pl.pallas_call(
        functools.partial(_assign_kernel, blk=blk),
        grid=(B, pb_steps),
        in_specs=[
            pl.BlockSpec((1, NG, 6), lambda b, p: (b, 0, 0)),
            pl.BlockSpec((4, blk), lambda b, p: (0, p)),
            pl.BlockSpec((1, 4, blk), lambda b, p: (b, 0, p)),
            pl.BlockSpec((1, NG, 1), lambda b, p: (b, 0, 0)),
        ],
        out_specs=pl.BlockSpec((1, 8, blk), lambda b, p: (b, 0, p)),
        out_shape=jax.ShapeDtypeStruct((B, 8, NPP), jnp.float32),
    )(gt_boxes, pr0_t, pr_t, best)

    rows = jnp.transpose(packed[:, :, :NP], (0, 2, 1)).reshape(B * NP, 8)

    blk3 = 8000
    cls_out, loc_true, amask = pl.pallas_call(
        _emit_kernel,
        grid=(B * NP // blk3,),
        in_specs=[pl.BlockSpec((blk3, 8), lambda i: (i, 0))],
        out_specs=[
            pl.BlockSpec((blk3, NC), lambda i: (i, 0)),
            pl.BlockSpec((blk3, 4), lambda i: (i, 0)),
            pl.BlockSpec((blk3, 1), lambda i: (i, 0)),
        ],
        out_shape=[
            jax.ShapeDtypeStruct((B * NP, NC), jnp.float32),
            jax.ShapeDtypeStruct((B * NP, 4), jnp.float32),
            jax.ShapeDtypeStruct((B * NP, 1), jnp.float32),
        ],
    )(rows)

    return (cls_out.reshape(B, NP, NC), loc_true.reshape(B, NP, 4),
            amask.reshape(B, NP, 1))
